# baseline (speedup 1.0000x reference)
import jax
import jax.numpy as jnp
from jax.experimental import pallas as pl
from jax.experimental.pallas import tpu as pltpu

_MARGIN = 0.5
_ALPHA = 0.5
_BSZ = 4096
_K = 8
_C_REAL = 1000
_CPAD = 1024
_CBLK = 256
_NBLK = _CPAD // _CBLK


def _topk_small(arr, k):
    jj16 = jax.lax.broadcasted_iota(jnp.int32, (16, _CBLK), 0)
    out = jnp.full((16, _CBLK), 4.0, jnp.float32)
    fill = jnp.zeros((1, _CBLK), jnp.int32)
    for _ in range(k - 1):
        v = jnp.min(arr, axis=0, keepdims=True)
        eq = arr == v
        m = jnp.sum(jnp.where(eq, 1, 0).astype(jnp.int32),
                    axis=0, keepdims=True)
        out = jnp.where((jj16 >= fill) & (jj16 < fill + m), v, out)
        arr = jnp.where(eq, 4.0, arr)
        fill = fill + m
    v = jnp.min(arr, axis=0, keepdims=True)
    return jnp.where(jj16 >= fill, v, out)


def _fold(arr):
    h = arr.shape[0] // 2
    lo = arr[:h]
    hi = arr[h:]
    return jnp.minimum(lo, hi), jnp.maximum(lo, hi)


def _topk_folded(arr, k):
    a1, b1 = _fold(arr)
    a2, b2 = _fold(a1)
    a3, b3 = _fold(a2)
    a4, b4 = _fold(a3)
    c1, d1 = _fold(b1)
    c2, d2 = _fold(b2)
    pad = jnp.full((32 - k - 20, _CBLK), 4.0, jnp.float32)
    cand = jnp.concatenate([
        _topk_small(a4, k)[0:k],
        _topk_small(b4, 4)[0:4],
        _topk_small(b3, 4)[0:4],
        _topk_small(c2, 4)[0:4],
        _topk_small(d2, 2)[0:2],
        _topk_small(c1, 4)[0:4],
        _topk_small(d1, 2)[0:2],
        pad], axis=0)
    return _topk_small(cand, k)


def _stats_kernel(x_ref, t_ref, out_ref, st_ref):
    i = pl.program_id(0)
    x = x_ref[...]
    t = t_ref[...]
    B = x.shape[0]
    Bf = jnp.float32(B)
    sig = jax.nn.sigmoid(x)
    posmask = t == 1.0
    bce_el = jnp.maximum(x, 0.0) - x * t - jnp.log(jnp.maximum(sig, 1.0 - sig))
    bce_col = jnp.sum(bce_el, axis=0, keepdims=True)
    counts = jnp.sum(t, axis=0, keepdims=True)
    npos = counts
    sp = jnp.where(posmask, sig, 0.0)
    s_pos = jnp.sum(sp, axis=0, keepdims=True)
    pos_pred = jnp.where(posmask, sig, 4.0)
    pv = _topk_folded(pos_pred, _K + 1)
    neg_pred = jnp.where(posmask, 4.0, sig)
    nv = _topk_folded(neg_pred, _K)

    p8 = jnp.zeros((1, _CBLK), jnp.float32)
    for j in range(_K):
        p8 = p8 + jnp.where(jnp.float32(j) < npos, pv[j:j + 1], 0.0)
    p9 = p8 + jnp.where(jnp.float32(_K) < npos, pv[_K:_K + 1], 0.0)
    pairsum = jnp.zeros((1, _CBLK), jnp.float32)
    for r in range(_K + 1):
        rg = jnp.float32(r) < npos
        for j in range(r + 1, _K + 1):
            g = rg & (jnp.float32(j) < npos)
            pairsum = pairsum + jnp.where(
                g, jnp.abs(pv[r:r + 1] - pv[j:j + 1]), 0.0)
    pairsum = pairsum + pairsum
    extra = jnp.where(npos > jnp.float32(_K + 1),
                      8.0 * (s_pos - p9) - (npos - 9.0) * p8, 0.0)
    cp = pairsum + extra

    n_n = jnp.minimum(jnp.float32(_K), Bf - npos)
    cn = jnp.zeros((1, _CBLK), jnp.float32)
    for j in range(_K):
        nvj = nv[j:j + 1]
        s_abs = jnp.sum(jnp.abs(sp - nvj), axis=0, keepdims=True)
        cn = cn + jnp.where(jnp.float32(j) < n_n,
                            s_abs - (Bf - npos) * nvj, 0.0)

    zero = jnp.zeros_like(counts)
    packed = jnp.concatenate(
        [counts, cp, cn, bce_col, zero, zero, zero, zero], axis=0)
    st_ref[pl.ds(i, 1), :, :] = packed[None]

    @pl.when(i == _NBLK - 1)
    def _combine():
        _combine_body(st_ref, out_ref)


def _combine_body(st_ref, out_ref):
    lane = jax.lax.broadcasted_iota(jnp.int32, (1, _CBLK), 1)
    counts_rows = []
    for b in range(_NBLK):
        c = st_ref[b, 0:1, :]
        real = (lane + b * _CBLK) < _C_REAL
        counts_rows.append(jnp.where(real, c, 0.0))
    src_cols = [
        jnp.transpose(jnp.broadcast_to(counts_rows[bs], (_CBLK, _CBLK)))[:, 0:1]
        for bs in range(_NBLK)]
    sidx0 = jax.lax.broadcasted_iota(jnp.int32, (_CBLK, _CBLK), 0)
    tidx0 = jax.lax.broadcasted_iota(jnp.int32, (_CBLK, _CBLK), 1)
    dp = jnp.zeros((1, _CBLK), jnp.float32)
    dn = jnp.zeros((1, _CBLK), jnp.float32)
    bce_acc = jnp.zeros((1, _CBLK), jnp.float32)
    any_sel = jnp.zeros((1, _CBLK), jnp.bool_)
    for bt in range(_NBLK):
        tgt = counts_rows[bt]
        acc = jnp.zeros((1, _CBLK), jnp.float32)
        for bs in range(_NBLK):
            sc = src_cols[bs]
            before = (sc < tgt) | ((sc == tgt)
                                   & (sidx0 + bs * _CBLK <= tidx0 + bt * _CBLK))
            acc = acc + jnp.sum(
                jnp.where(before, jnp.broadcast_to(sc, (_CBLK, _CBLK)), 0.0),
                axis=0, keepdims=True)
        real = (lane + bt * _CBLK) < _C_REAL
        sel = (acc <= 0.5 * _BSZ) & (tgt > 1.0) & real
        n_n = jnp.minimum(jnp.float32(_K), jnp.float32(_BSZ) - tgt)
        n_p = jnp.minimum(jnp.float32(_K), tgt - 1.0)
        col_valid = sel & (n_n >= 1.0)
        dp = dp + jnp.where(col_valid, n_n * st_ref[bt, 1:2, :], 0.0)
        dn = dn + jnp.where(col_valid, n_p * st_ref[bt, 2:3, :], 0.0)
        bce_acc = bce_acc + jnp.where(real, st_ref[bt, 3:4, :], 0.0)
        any_sel = any_sel | sel
    bce = jnp.sum(bce_acc) / jnp.float32(_BSZ * _C_REAL)
    crl = jnp.maximum(jnp.sum(dp) - jnp.sum(dn) + _MARGIN, 0.0)
    has_trip = jnp.any(any_sel)
    res = jnp.where(has_trip, _ALPHA * crl + (1.0 - _ALPHA) * bce, bce)
    out_ref[...] = res.reshape(1, 1)


@jax.jit
def _impl(x, t):
    out = pl.pallas_call(
        _stats_kernel,
        grid=(_NBLK,),
        in_specs=[pl.BlockSpec((_BSZ, _CBLK), lambda i: (0, i)),
                  pl.BlockSpec((_BSZ, _CBLK), lambda i: (0, i))],
        out_specs=pl.BlockSpec((1, 1), lambda i: (0, 0)),
        out_shape=jax.ShapeDtypeStruct((1, 1), jnp.float32),
        scratch_shapes=[pltpu.VMEM((_NBLK, 8, _CBLK), jnp.float32)],
        compiler_params=pltpu.CompilerParams(
            dimension_semantics=("arbitrary",)),
    )(x, t)
    return out[0, 0]


def kernel(input, target, X):
    return _impl(input, target)


# speedup vs baseline: 1.0562x; 1.0562x over previous
import jax
import jax.numpy as jnp
from jax.experimental import pallas as pl
from jax.experimental.pallas import tpu as pltpu

_MARGIN = 0.5
_ALPHA = 0.5
_BSZ = 4096
_K = 8
_C_REAL = 1000
_CPAD = 1024
_CBLK = 128
_NBLK = _CPAD // _CBLK


def _topk_small(arr, k):
    jj16 = jax.lax.broadcasted_iota(jnp.int32, (16, _CBLK), 0)
    out = jnp.full((16, _CBLK), 4.0, jnp.float32)
    fill = jnp.zeros((1, _CBLK), jnp.int32)
    for _ in range(k - 1):
        v = jnp.min(arr, axis=0, keepdims=True)
        eq = arr == v
        m = jnp.sum(jnp.where(eq, 1, 0).astype(jnp.int32),
                    axis=0, keepdims=True)
        out = jnp.where((jj16 >= fill) & (jj16 < fill + m), v, out)
        arr = jnp.where(eq, 4.0, arr)
        fill = fill + m
    v = jnp.min(arr, axis=0, keepdims=True)
    return jnp.where(jj16 >= fill, v, out)


def _fold(arr):
    h = arr.shape[0] // 2
    lo = arr[:h]
    hi = arr[h:]
    return jnp.minimum(lo, hi), jnp.maximum(lo, hi)


def _topk_folded(arr, k):
    a1, b1 = _fold(arr)
    a2, b2 = _fold(a1)
    a3, b3 = _fold(a2)
    a4, b4 = _fold(a3)
    c1, d1 = _fold(b1)
    c2, d2 = _fold(b2)
    pad = jnp.full((32 - k - 20, _CBLK), 4.0, jnp.float32)
    cand = jnp.concatenate([
        _topk_small(a4, k)[0:k],
        _topk_small(b4, 4)[0:4],
        _topk_small(b3, 4)[0:4],
        _topk_small(c2, 4)[0:4],
        _topk_small(d2, 2)[0:2],
        _topk_small(c1, 4)[0:4],
        _topk_small(d1, 2)[0:2],
        pad], axis=0)
    return _topk_small(cand, k)


def _stats_kernel(x_ref, t_ref, out_ref, st_ref):
    i = pl.program_id(0)
    x = x_ref[...]
    t = t_ref[...]
    B = x.shape[0]
    Bf = jnp.float32(B)
    sig = jax.nn.sigmoid(x)
    posmask = t == 1.0
    bce_el = jnp.maximum(x, 0.0) - x * t - jnp.log(jnp.maximum(sig, 1.0 - sig))
    bce_col = jnp.sum(bce_el, axis=0, keepdims=True)
    counts = jnp.sum(t, axis=0, keepdims=True)
    npos = counts
    sp = jnp.where(posmask, sig, 0.0)
    s_pos = jnp.sum(sp, axis=0, keepdims=True)
    pos_pred = jnp.where(posmask, sig, 4.0)
    pv = _topk_folded(pos_pred, _K + 1)
    neg_pred = jnp.where(posmask, 4.0, sig)
    nv = _topk_folded(neg_pred, _K)

    p8 = jnp.zeros((1, _CBLK), jnp.float32)
    for j in range(_K):
        p8 = p8 + jnp.where(jnp.float32(j) < npos, pv[j:j + 1], 0.0)
    p9 = p8 + jnp.where(jnp.float32(_K) < npos, pv[_K:_K + 1], 0.0)
    pairsum = jnp.zeros((1, _CBLK), jnp.float32)
    for r in range(_K + 1):
        rg = jnp.float32(r) < npos
        for j in range(r + 1, _K + 1):
            g = rg & (jnp.float32(j) < npos)
            pairsum = pairsum + jnp.where(
                g, jnp.abs(pv[r:r + 1] - pv[j:j + 1]), 0.0)
    pairsum = pairsum + pairsum
    extra = jnp.where(npos > jnp.float32(_K + 1),
                      8.0 * (s_pos - p9) - (npos - 9.0) * p8, 0.0)
    cp = pairsum + extra

    n_n = jnp.minimum(jnp.float32(_K), Bf - npos)
    cn = jnp.zeros((1, _CBLK), jnp.float32)
    for j in range(_K):
        nvj = nv[j:j + 1]
        s_abs = jnp.sum(jnp.abs(sp - nvj), axis=0, keepdims=True)
        cn = cn + jnp.where(jnp.float32(j) < n_n,
                            s_abs - (Bf - npos) * nvj, 0.0)

    zero = jnp.zeros_like(counts)
    packed = jnp.concatenate(
        [counts, cp, cn, bce_col, zero, zero, zero, zero], axis=0)
    st_ref[pl.ds(i, 1), :, :] = packed[None]

    @pl.when(i == _NBLK - 1)
    def _combine():
        _combine_body(st_ref, out_ref)


def _combine_body(st_ref, out_ref):
    lane = jax.lax.broadcasted_iota(jnp.int32, (1, _CBLK), 1)
    counts_rows = []
    for b in range(_NBLK):
        c = st_ref[b, 0:1, :]
        real = (lane + b * _CBLK) < _C_REAL
        counts_rows.append(jnp.where(real, c, 0.0))
    src_cols = [
        jnp.transpose(jnp.broadcast_to(counts_rows[bs], (_CBLK, _CBLK)))[:, 0:1]
        for bs in range(_NBLK)]
    sidx0 = jax.lax.broadcasted_iota(jnp.int32, (_CBLK, _CBLK), 0)
    tidx0 = jax.lax.broadcasted_iota(jnp.int32, (_CBLK, _CBLK), 1)
    dp = jnp.zeros((1, _CBLK), jnp.float32)
    dn = jnp.zeros((1, _CBLK), jnp.float32)
    bce_acc = jnp.zeros((1, _CBLK), jnp.float32)
    any_sel = jnp.zeros((1, _CBLK), jnp.bool_)
    for bt in range(_NBLK):
        tgt = counts_rows[bt]
        acc = jnp.zeros((1, _CBLK), jnp.float32)
        for bs in range(_NBLK):
            sc = src_cols[bs]
            before = (sc < tgt) | ((sc == tgt)
                                   & (sidx0 + bs * _CBLK <= tidx0 + bt * _CBLK))
            acc = acc + jnp.sum(
                jnp.where(before, jnp.broadcast_to(sc, (_CBLK, _CBLK)), 0.0),
                axis=0, keepdims=True)
        real = (lane + bt * _CBLK) < _C_REAL
        sel = (acc <= 0.5 * _BSZ) & (tgt > 1.0) & real
        n_n = jnp.minimum(jnp.float32(_K), jnp.float32(_BSZ) - tgt)
        n_p = jnp.minimum(jnp.float32(_K), tgt - 1.0)
        col_valid = sel & (n_n >= 1.0)
        dp = dp + jnp.where(col_valid, n_n * st_ref[bt, 1:2, :], 0.0)
        dn = dn + jnp.where(col_valid, n_p * st_ref[bt, 2:3, :], 0.0)
        bce_acc = bce_acc + jnp.where(real, st_ref[bt, 3:4, :], 0.0)
        any_sel = any_sel | sel
    bce = jnp.sum(bce_acc) / jnp.float32(_BSZ * _C_REAL)
    crl = jnp.maximum(jnp.sum(dp) - jnp.sum(dn) + _MARGIN, 0.0)
    has_trip = jnp.any(any_sel)
    res = jnp.where(has_trip, _ALPHA * crl + (1.0 - _ALPHA) * bce, bce)
    out_ref[...] = res.reshape(1, 1)


@jax.jit
def _impl(x, t):
    out = pl.pallas_call(
        _stats_kernel,
        grid=(_NBLK,),
        in_specs=[pl.BlockSpec((_BSZ, _CBLK), lambda i: (0, i)),
                  pl.BlockSpec((_BSZ, _CBLK), lambda i: (0, i))],
        out_specs=pl.BlockSpec((1, 1), lambda i: (0, 0)),
        out_shape=jax.ShapeDtypeStruct((1, 1), jnp.float32),
        scratch_shapes=[pltpu.VMEM((_NBLK, 8, _CBLK), jnp.float32)],
        compiler_params=pltpu.CompilerParams(
            dimension_semantics=("arbitrary",)),
    )(x, t)
    return out[0, 0]


def kernel(input, target, X):
    return _impl(input, target)
